# baseline (speedup 1.0000x reference)
import functools
import jax
import jax.numpy as jnp
from jax import lax
from jax.experimental import pallas as pl
from jax.experimental.pallas import tpu as pltpu
from jax.experimental.pallas import tpu_sc as plsc

D_MODEL = 64
HD = 32
SCALE = 8.0

NC, NS = 2, 16
NW = NC * NS
RG = 128
G = 5
CH = G * RG

B = 4096 * 200
B_PER_W = B // NW
GROUPS_PER_W = B_PER_W // RG
CHUNKS_PER_W = B_PER_W // CH


def _emb_body(x_hbm, lutA_hbm, lutB_hbm, out_hbm, idx_v,
              bufA0, bufA1, bufB0, bufB1, gsem0, gsem1):
    wid = lax.axis_index("s") * NC + lax.axis_index("c")
    gbase = wid * GROUPS_PER_W
    pltpu.sync_copy(x_hbm.at[pl.ds(gbase, GROUPS_PER_W)], idx_v)

    bufsA = (bufA0, bufA1)
    bufsB = (bufB0, bufB1)
    sems = (gsem0, gsem1)

    def fire(chunk, b):
        for j in range(G):
            idx = idx_v.at[chunk * G + j]
            sl = pl.ds(j * RG, RG)
            pltpu.async_copy(lutA_hbm.at[idx], bufsA[b].at[sl], sems[b])
            pltpu.async_copy(lutB_hbm.at[idx], bufsB[b].at[sl], sems[b])

    def drain(b):
        pltpu.make_async_copy(lutA_hbm.at[pl.ds(0, CH)], bufsA[b], sems[b]).wait()
        pltpu.make_async_copy(lutB_hbm.at[pl.ds(0, CH)], bufsB[b], sems[b]).wait()

    fire(0, 0)

    @pl.loop(0, CHUNKS_PER_W, step=2)
    def _chunk(g):
        for b in range(2):
            chunk = g + b

            @pl.when(chunk + 1 < CHUNKS_PER_W)
            def _():
                fire(chunk + 1, 1 - b)

            drain(b)

            @pl.loop(0, CH, unroll=8)
            def _row(i):
                for j in range(HD // 16):
                    sl = pl.ds(j * 16, 16)
                    bufsA[b][i, sl] = bufsA[b][i, sl] * SCALE
                    bufsB[b][i, sl] = bufsB[b][i, sl] * SCALE

            crow = (gbase + chunk * G) * RG
            rows = pl.ds(crow, CH)
            pltpu.sync_copy(bufsA[b], out_hbm.at[rows, pl.ds(0, HD)])
            pltpu.sync_copy(bufsB[b], out_hbm.at[rows, pl.ds(HD, HD)])


@jax.jit
def _emb(x2, lutA, lutB):
    mesh = plsc.VectorSubcoreMesh(
        core_axis_name="c", subcore_axis_name="s", num_cores=NC, num_subcores=NS
    )
    run = pl.kernel(
        _emb_body,
        out_type=jax.ShapeDtypeStruct((B, 128), jnp.float32),
        mesh=mesh,
        scratch_types=[
            pltpu.VMEM((GROUPS_PER_W, RG), jnp.int32),
            pltpu.VMEM((CH, HD), jnp.float32),
            pltpu.VMEM((CH, HD), jnp.float32),
            pltpu.VMEM((CH, HD), jnp.float32),
            pltpu.VMEM((CH, HD), jnp.float32),
            pltpu.SemaphoreType.DMA,
            pltpu.SemaphoreType.DMA,
        ],
        compiler_params=pltpu.CompilerParams(
            use_tc_tiling_on_sc=False, needs_layout_passes=False
        ),
    )
    return run(x2, lutA, lutB)


def kernel(x, lut):
    x2 = x.reshape(B // RG, RG).astype(jnp.int32)
    out = _emb(x2, lut[:, :HD], lut[:, HD:])
    return out[:, :D_MODEL].reshape(x.shape[0], x.shape[1], D_MODEL)


# speedup vs baseline: 2.2655x; 2.2655x over previous
import functools
import jax
import jax.numpy as jnp
from jax import lax
from jax.experimental import pallas as pl
from jax.experimental.pallas import tpu as pltpu
from jax.experimental.pallas import tpu_sc as plsc

D_MODEL = 64
SCALE = 8.0

NC, NS = 2, 16
NW = NC * NS
RG = 128
G = 5
CH = G * RG

B = 4096 * 200
B_PER_W = B // NW
GROUPS_PER_W = B_PER_W // RG
CHUNKS_PER_W = B_PER_W // CH


def _emb_body(x_hbm, lut_hbm, out_hbm, idx_v, buf0, buf1, gsem0, gsem1):
    wid = lax.axis_index("s") * NC + lax.axis_index("c")
    gbase = wid * GROUPS_PER_W
    pltpu.sync_copy(x_hbm.at[pl.ds(gbase, GROUPS_PER_W)], idx_v)

    bufs = (buf0, buf1)
    sems = (gsem0, gsem1)

    def fire(chunk, buf, sem):
        for j in range(G):
            pltpu.async_copy(
                lut_hbm.at[idx_v.at[chunk * G + j]],
                buf.at[pl.ds(j * RG, RG)],
                sem,
            )

    def drain(buf, sem):
        pltpu.make_async_copy(lut_hbm.at[pl.ds(0, CH)], buf, sem).wait()

    fire(0, buf0, gsem0)

    @pl.loop(0, CHUNKS_PER_W, step=2)
    def _chunk(g):
        for b in range(2):
            chunk = g + b
            buf, sem = bufs[b], sems[b]

            @pl.when(chunk + 1 < CHUNKS_PER_W)
            def _():
                fire(chunk + 1, bufs[1 - b], sems[1 - b])

            drain(buf, sem)

            @pl.loop(0, CH, unroll=8)
            def _row(i):
                for j in range(D_MODEL // 16):
                    sl = pl.ds(j * 16, 16)
                    buf[i, sl] = buf[i, sl] * SCALE

            crow = (gbase + chunk * G) * RG
            pltpu.sync_copy(buf, out_hbm.at[pl.ds(crow, CH), pl.ds(0, D_MODEL)])


@jax.jit
def _emb(x2, lut):
    mesh = plsc.VectorSubcoreMesh(
        core_axis_name="c", subcore_axis_name="s", num_cores=NC, num_subcores=NS
    )
    run = pl.kernel(
        _emb_body,
        out_type=jax.ShapeDtypeStruct((B, 128), jnp.float32),
        mesh=mesh,
        scratch_types=[
            pltpu.VMEM((GROUPS_PER_W, RG), jnp.int32),
            pltpu.VMEM((CH, D_MODEL), jnp.float32),
            pltpu.VMEM((CH, D_MODEL), jnp.float32),
            pltpu.SemaphoreType.DMA,
            pltpu.SemaphoreType.DMA,
        ],
        compiler_params=pltpu.CompilerParams(
            use_tc_tiling_on_sc=False, needs_layout_passes=False
        ),
    )
    return run(x2, lut)


def kernel(x, lut):
    x2 = x.reshape(B // RG, RG).astype(jnp.int32)
    out = _emb(x2, lut)
    return out[:, :D_MODEL].reshape(x.shape[0], x.shape[1], D_MODEL)
